# baseline (speedup 1.0000x reference)
import functools

import jax
import jax.numpy as jnp
from jax import lax
from jax.experimental import pallas as pl
from jax.experimental.pallas import tpu as pltpu
from jax.experimental.pallas import tpu_sc as plsc

B = 16384
D = 32
H = 256
NT = 6
NBIG = 2
NSML = 4
NC, NS = 2, 16
NW = NC * NS
BPW = B // NW
CH = 128
NCH = BPW // CH
NRING = 12


def _sc_gather_big(cat0, cat1, tt0, tt1, bh):
  mesh = plsc.VectorSubcoreMesh(core_axis_name="c", subcore_axis_name="s")
  bpw = bh // NW

  @functools.partial(
      pl.kernel,
      out_type=jax.ShapeDtypeStruct((NBIG, bh, D), jnp.float32),
      mesh=mesh,
      compiler_params=pltpu.CompilerParams(
          use_tc_tiling_on_sc=True, needs_layout_passes=False),
      scratch_types=[
          pltpu.VMEM((bpw + 16,), jnp.int32),
          pltpu.VMEM((NRING, 4, 8, 128), jnp.float32),
          pltpu.VMEM((bpw, D), jnp.float32),
          pltpu.SemaphoreType.DMA,
      ],
  )
  def k(c0, c1, t0, t1, out, idx_v, tile_v, emb_v, sem):
    wid = lax.axis_index("s") * NC + lax.axis_index("c")
    base = wid * bpw
    d16 = lax.iota(jnp.int32, 16)
    ga, ra = d16 // 8, d16 % 8
    gb, rb = ga + 2, ra

    for t, (cat, tab) in enumerate(((c0, t0), (c1, t1))):
      pltpu.sync_copy(cat.at[pl.ds(base, bpw)], idx_v.at[pl.ds(0, bpw)])

      def fire(b, tab=tab):
        v = idx_v[pl.ds(b, 16)][0]
        pltpu.async_copy(
            tab.at[:, :, pl.ds((v // 128) * 128, 128)],
            tile_v.at[lax.rem(b, NRING)], sem)

      for b0 in range(NRING):
        fire(b0)

      def body(b, carry, tab=tab):
        slot = lax.rem(b, NRING)
        pltpu.make_async_copy(
            tab.at[:, :, pl.ds(0, 128)], tile_v.at[slot], sem).wait()
        v = idx_v[pl.ds(b, 16)][0]
        lane = jnp.full((16,), lax.rem(v, 128), jnp.int32)
        bsp = jnp.full((16,), b, jnp.int32)
        va = plsc.load_gather(tile_v.at[slot], [ga, ra, lane])
        vb = plsc.load_gather(tile_v.at[slot], [gb, rb, lane])
        plsc.store_scatter(emb_v, [bsp, d16], va)
        plsc.store_scatter(emb_v, [bsp, d16 + 16], vb)

        @pl.when(b + NRING < bpw)
        def _():
          fire(b + NRING)

        return carry

      lax.fori_loop(0, bpw, body, 0)
      pltpu.sync_copy(emb_v, out.at[t, pl.ds(base, bpw)])

  return k(cat0, cat1, tt0, tt1)


def _sc_gather_small(cats2d, tables):
  mesh = plsc.VectorSubcoreMesh(core_axis_name="c", subcore_axis_name="s")

  @functools.partial(
      pl.kernel,
      out_type=jax.ShapeDtypeStruct((NSML, B, D), jnp.float32),
      mesh=mesh,
      compiler_params=pltpu.CompilerParams(use_tc_tiling_on_sc=False),
      scratch_types=[
          pltpu.VMEM((NSML * NCH, CH), jnp.int32),
          pltpu.VMEM((NSML, BPW, D), jnp.float32),
          pltpu.SemaphoreType.DMA,
      ],
  )
  def k(c0, c1, c2, c3, t0, t1, t2, t3, out, idx_v, rows_v, sem):
    wid = lax.axis_index("s") * NC + lax.axis_index("c")
    base = wid * BPW
    cats = [c0, c1, c2, c3]
    tabs = [t0, t1, t2, t3]
    copies = []
    for i in range(NSML):
      pltpu.sync_copy(cats[i].at[pl.ds(wid * NCH, NCH)],
                      idx_v.at[pl.ds(i * NCH, NCH)])
      for j in range(NCH):
        copies.append(pltpu.async_copy(
            tabs[i].at[idx_v.at[i * NCH + j]],
            rows_v.at[i, pl.ds(j * CH, CH)], sem))
    for i in range(NSML):
      for j in range(NCH):
        copies[i * NCH + j].wait()
      pltpu.sync_copy(rows_v.at[i], out.at[i, pl.ds(base, BPW)])

  return k(*cats2d, *tables)


def _mlp_body(xb_ref, xs_ref, n_ref, w1b_ref, w1s_ref, wa_ref, w2_ref,
              b2_ref, o_ref):
  h = jnp.dot(n_ref[...], wa_ref[...], preferred_element_type=jnp.float32)
  for t in range(NBIG):
    h = h + jnp.dot(xb_ref[t], w1b_ref[t],
                    preferred_element_type=jnp.float32)
  for t in range(NSML):
    h = h + jnp.dot(xs_ref[t], w1s_ref[t],
                    preferred_element_type=jnp.float32)
  h = jnp.maximum(h, 0.0)
  o_ref[...] = (
      jnp.dot(h, w2_ref[...], preferred_element_type=jnp.float32)
      + b2_ref[...]
  )


def _tc_mlp(xb, xs, num_aug, w1b, w1s, w1_aug, w2, b2_2d, bh):
  blk = 2048
  nb = bh // blk
  return pl.pallas_call(
      _mlp_body,
      grid=(nb,),
      in_specs=[
          pl.BlockSpec((NBIG, blk, D), lambda i: (0, i, 0)),
          pl.BlockSpec((NSML, blk, D), lambda i: (0, i, 0)),
          pl.BlockSpec((blk, 8), lambda i: (i, 0)),
          pl.BlockSpec((NBIG, D, H), lambda i: (0, 0, 0)),
          pl.BlockSpec((NSML, D, H), lambda i: (0, 0, 0)),
          pl.BlockSpec((8, H), lambda i: (0, 0)),
          pl.BlockSpec((H, D), lambda i: (0, 0)),
          pl.BlockSpec((1, D), lambda i: (0, 0)),
      ],
      out_specs=pl.BlockSpec((blk, D), lambda i: (i, 0)),
      out_shape=jax.ShapeDtypeStruct((bh, D), jnp.float32),
  )(xb, xs, num_aug, w1b, w1s, w1_aug, w2, b2_2d)


def kernel(cat0, cat1, cat2, cat3, cat4, cat5, numerical_inputs,
           T0, T1, T2, T3, T4, T5, W1, b1, W2, b2):
  cb = [c.astype(jnp.int32) for c in (cat0, cat1)]
  cs = [c.astype(jnp.int32).reshape(NW * NCH, CH)
        for c in (cat2, cat3, cat4, cat5)]
  tt0 = T0.T.reshape(4, 8, T0.shape[0])
  tt1 = T1.T.reshape(4, 8, T1.shape[0])
  emb_sml = _sc_gather_small(cs, [T2, T3, T4, T5])

  ones = jnp.ones((B, 1), jnp.float32)
  zeros = jnp.zeros((B, 3), jnp.float32)
  num_aug = jnp.concatenate([numerical_inputs, ones, zeros], axis=1)
  w1_aug = jnp.concatenate(
      [W1[NT * D:], b1[None, :], jnp.zeros((3, H), jnp.float32)], axis=0)
  w1b = W1[:NBIG * D].reshape(NBIG, D, H)
  w1s = W1[NBIG * D:NT * D].reshape(NSML, D, H)

  bh = B // 2
  outs = []
  for hfi in range(2):
    sl = slice(hfi * bh, (hfi + 1) * bh)
    ebh = _sc_gather_big(cb[0][sl], cb[1][sl], tt0, tt1, bh)
    outs.append(_tc_mlp(
        ebh, lax.slice_in_dim(emb_sml, hfi * bh, (hfi + 1) * bh, axis=1),
        num_aug[sl], w1b, w1s, w1_aug, W2, b2[None, :], bh))
  return jnp.concatenate(outs, axis=0)


# speedup vs baseline: 1.0603x; 1.0603x over previous
import functools

import jax
import jax.numpy as jnp
from jax import lax
from jax.experimental import pallas as pl
from jax.experimental.pallas import tpu as pltpu
from jax.experimental.pallas import tpu_sc as plsc

B = 16384
D = 32
H = 256
NT = 6
NBIG = 2
NSML = 4
NC, NS = 2, 16
NW = NC * NS
BPW = B // NW
CH = 128
NCH = BPW // CH
NRING = 12


def _sc_gather_big(cat0, cat1, tt0, tt1):
  mesh = plsc.VectorSubcoreMesh(core_axis_name="c", subcore_axis_name="s")

  @functools.partial(
      pl.kernel,
      out_type=jax.ShapeDtypeStruct((NBIG, B, D), jnp.float32),
      mesh=mesh,
      compiler_params=pltpu.CompilerParams(
          use_tc_tiling_on_sc=True, needs_layout_passes=False),
      scratch_types=[
          pltpu.VMEM((BPW + 16,), jnp.int32),
          pltpu.VMEM((NRING, 4, 8, 128), jnp.float32),
          pltpu.VMEM((BPW, D), jnp.float32),
          pltpu.SemaphoreType.DMA,
      ],
  )
  def k(c0, c1, t0, t1, out, idx_v, tile_v, emb_v, sem):
    wid = lax.axis_index("s") * NC + lax.axis_index("c")
    base = wid * BPW
    d16 = lax.iota(jnp.int32, 16)
    ga, ra = d16 // 8, d16 % 8
    gb, rb = ga + 2, ra

    for t, (cat, tab) in enumerate(((c0, t0), (c1, t1))):
      pltpu.sync_copy(cat.at[pl.ds(base, BPW)], idx_v.at[pl.ds(0, BPW)])

      def fire(b, tab=tab):
        v = idx_v[pl.ds(b, 16)][0]
        pltpu.async_copy(
            tab.at[:, :, pl.ds((v // 128) * 128, 128)],
            tile_v.at[lax.rem(b, NRING)], sem)

      for b0 in range(NRING):
        fire(b0)

      def body(b, carry, tab=tab):
        slot = lax.rem(b, NRING)
        pltpu.make_async_copy(
            tab.at[:, :, pl.ds(0, 128)], tile_v.at[slot], sem).wait()
        v = idx_v[pl.ds(b, 16)][0]
        lane = jnp.full((16,), lax.rem(v, 128), jnp.int32)
        bsp = jnp.full((16,), b, jnp.int32)
        va = plsc.load_gather(tile_v.at[slot], [ga, ra, lane])
        vb = plsc.load_gather(tile_v.at[slot], [gb, rb, lane])
        plsc.store_scatter(emb_v, [bsp, d16], va)
        plsc.store_scatter(emb_v, [bsp, d16 + 16], vb)

        @pl.when(b + NRING < BPW)
        def _():
          fire(b + NRING)

        return carry

      lax.fori_loop(0, BPW, body, 0)
      pltpu.sync_copy(emb_v, out.at[t, pl.ds(base, BPW)])

  return k(cat0, cat1, tt0, tt1)


def _sc_gather_small(cats2d, tables):
  mesh = plsc.VectorSubcoreMesh(core_axis_name="c", subcore_axis_name="s")

  @functools.partial(
      pl.kernel,
      out_type=jax.ShapeDtypeStruct((NSML, B, D), jnp.float32),
      mesh=mesh,
      compiler_params=pltpu.CompilerParams(use_tc_tiling_on_sc=False),
      scratch_types=[
          pltpu.VMEM((NSML * NCH, CH), jnp.int32),
          pltpu.VMEM((NSML, BPW, D), jnp.float32),
          pltpu.SemaphoreType.DMA,
      ],
  )
  def k(c0, c1, c2, c3, t0, t1, t2, t3, out, idx_v, rows_v, sem):
    wid = lax.axis_index("s") * NC + lax.axis_index("c")
    base = wid * BPW
    cats = [c0, c1, c2, c3]
    tabs = [t0, t1, t2, t3]
    copies = []
    for i in range(NSML):
      pltpu.sync_copy(cats[i].at[pl.ds(wid * NCH, NCH)],
                      idx_v.at[pl.ds(i * NCH, NCH)])
      for j in range(NCH):
        copies.append(pltpu.async_copy(
            tabs[i].at[idx_v.at[i * NCH + j]],
            rows_v.at[i, pl.ds(j * CH, CH)], sem))
    for i in range(NSML):
      for j in range(NCH):
        copies[i * NCH + j].wait()
      pltpu.sync_copy(rows_v.at[i], out.at[i, pl.ds(base, BPW)])

  return k(*cats2d, *tables)


def _mlp_body(xb_ref, xs_ref, n_ref, w1b_ref, w1s_ref, wa_ref, w2_ref,
              b2_ref, o_ref):
  h = jnp.dot(n_ref[...], wa_ref[...], preferred_element_type=jnp.float32)
  for t in range(NBIG):
    h = h + jnp.dot(xb_ref[t], w1b_ref[t],
                    preferred_element_type=jnp.float32)
  for t in range(NSML):
    h = h + jnp.dot(xs_ref[t], w1s_ref[t],
                    preferred_element_type=jnp.float32)
  h = jnp.maximum(h, 0.0)
  o_ref[...] = (
      jnp.dot(h, w2_ref[...], preferred_element_type=jnp.float32)
      + b2_ref[...]
  )


def _tc_mlp(xb, xs, num_aug, w1b, w1s, w1_aug, w2, b2_2d):
  blk = 2048
  nb = B // blk
  return pl.pallas_call(
      _mlp_body,
      grid=(nb,),
      in_specs=[
          pl.BlockSpec((NBIG, blk, D), lambda i: (0, i, 0)),
          pl.BlockSpec((NSML, blk, D), lambda i: (0, i, 0)),
          pl.BlockSpec((blk, 8), lambda i: (i, 0)),
          pl.BlockSpec((NBIG, D, H), lambda i: (0, 0, 0)),
          pl.BlockSpec((NSML, D, H), lambda i: (0, 0, 0)),
          pl.BlockSpec((8, H), lambda i: (0, 0)),
          pl.BlockSpec((H, D), lambda i: (0, 0)),
          pl.BlockSpec((1, D), lambda i: (0, 0)),
      ],
      out_specs=pl.BlockSpec((blk, D), lambda i: (i, 0)),
      out_shape=jax.ShapeDtypeStruct((B, D), jnp.float32),
  )(xb, xs, num_aug, w1b, w1s, w1_aug, w2, b2_2d)


def kernel(cat0, cat1, cat2, cat3, cat4, cat5, numerical_inputs,
           T0, T1, T2, T3, T4, T5, W1, b1, W2, b2):
  cb = [c.astype(jnp.int32) for c in (cat0, cat1)]
  cs = [c.astype(jnp.int32).reshape(NW * NCH, CH)
        for c in (cat2, cat3, cat4, cat5)]
  tt0 = T0.T.reshape(4, 8, T0.shape[0])
  tt1 = T1.T.reshape(4, 8, T1.shape[0])
  emb_big = _sc_gather_big(cb[0], cb[1], tt0, tt1)
  emb_sml = _sc_gather_small(cs, [T2, T3, T4, T5])

  ones = jnp.ones((B, 1), jnp.float32)
  zeros = jnp.zeros((B, 3), jnp.float32)
  num_aug = jnp.concatenate([numerical_inputs, ones, zeros], axis=1)
  w1_aug = jnp.concatenate(
      [W1[NT * D:], b1[None, :], jnp.zeros((3, H), jnp.float32)], axis=0)
  w1b = W1[:NBIG * D].reshape(NBIG, D, H)
  w1s = W1[NBIG * D:NT * D].reshape(NSML, D, H)
  return _tc_mlp(emb_big, emb_sml, num_aug, w1b, w1s, w1_aug, W2,
                 b2[None, :])
